# Initial kernel scaffold; baseline (speedup 1.0000x reference)
import jax
import jax.numpy as jnp
from jax.experimental import pallas as pl


def kernel(feature, edge_index, W, b):
    raise NotImplementedError("write your pallas kernel here")


# speedup vs baseline: 6.1375x; 6.1375x over previous
import functools

import jax
import jax.numpy as jnp
from jax import lax
from jax.experimental import pallas as pl
from jax.experimental.pallas import tpu as pltpu
from jax.experimental.pallas import tpu_sc as plsc

D = 128
NC = 2
NS = 16
NW = NC * NS
CHUNK = 128
LANES = 16


def _sc_aggregate(feat_pad, src_w, dst_w, zin, n_pad):
    kch = src_w.shape[1]
    rows_per_tile = n_pad // NS

    mesh = plsc.VectorSubcoreMesh(core_axis_name="c", subcore_axis_name="s")

    @functools.partial(
        pl.kernel,
        mesh=mesh,
        compiler_params=pltpu.CompilerParams(needs_layout_passes=False),
        out_type=[
            jax.ShapeDtypeStruct((NC, n_pad, D), jnp.float32),
            jax.ShapeDtypeStruct((NW, n_pad), jnp.float32),
        ],
        scratch_types=[
            pltpu.VMEM((kch, CHUNK), jnp.int32),
            pltpu.VMEM((kch, CHUNK), jnp.int32),
            pltpu.VMEM((CHUNK, D), jnp.float32),
            pltpu.VMEM((n_pad,), jnp.float32),
            pltpu.VMEM_SHARED((n_pad, D), jnp.float32),
            pltpu.SemaphoreType.DMA,
        ],
    )
    def k(feat_hbm, src_hbm, dst_hbm, zin_hbm, agg_hbm, deg_hbm,
          src_v, dst_v, rows_v, deg_v, acc_sh, sem):
        cid = lax.axis_index("c")
        sid = lax.axis_index("s")
        wid = sid * NC + cid
        r0 = sid * rows_per_tile

        pltpu.sync_copy(zin_hbm, rows_v)
        for t in range(rows_per_tile // CHUNK):
            pltpu.sync_copy(rows_v, acc_sh.at[pl.ds(r0 + t * CHUNK, CHUNK)])

        zeros16 = jnp.zeros((LANES,), jnp.float32)

        def zbody(i, carry):
            deg_v[pl.ds(i * LANES, LANES)] = zeros16
            return carry

        lax.fori_loop(0, n_pad // LANES, zbody, 0)

        pltpu.sync_copy(src_hbm.at[wid], src_v)
        pltpu.sync_copy(dst_hbm.at[wid], dst_v)
        plsc.subcore_barrier()

        ones16 = jnp.full((LANES,), 1.0, jnp.float32)

        def body(j, carry):
            pltpu.async_copy(feat_hbm.at[src_v.at[j]], rows_v, sem).wait()
            pltpu.sync_copy(rows_v, acc_sh.at[dst_v.at[j]], add=True)
            for i in range(CHUNK // LANES):
                idx = dst_v[j, pl.ds(i * LANES, LANES)]
                plsc.addupdate_scatter(deg_v, [idx], ones16)
            return carry

        lax.fori_loop(0, kch, body, 0)
        plsc.subcore_barrier()

        pltpu.sync_copy(acc_sh.at[pl.ds(r0, rows_per_tile)],
                        agg_hbm.at[cid].at[pl.ds(r0, rows_per_tile)])
        pltpu.sync_copy(deg_v, deg_hbm.at[wid])

    return k(feat_pad, src_w, dst_w, zin)


def _tc_dense(agg2, deg_t, feat_pad, wt, b2, n_pad):
    blk = 1024
    grid = (n_pad // blk,)

    def body(agg_ref, deg_ref, f_ref, w_ref, b_ref, o_ref):
        a = agg_ref[0] + agg_ref[1]
        deg = jnp.sum(deg_ref[...], axis=1, keepdims=True)
        h = jnp.where(deg > 0.0, a / jnp.maximum(deg, 1.0), f_ref[...])
        y = jnp.dot(h, w_ref[...], preferred_element_type=jnp.float32)
        o_ref[...] = jnp.maximum(y + b_ref[...], 0.0)

    return pl.pallas_call(
        body,
        grid=grid,
        in_specs=[
            pl.BlockSpec((NC, blk, D), lambda i: (0, i, 0)),
            pl.BlockSpec((blk, NW), lambda i: (i, 0)),
            pl.BlockSpec((blk, D), lambda i: (i, 0)),
            pl.BlockSpec((D, D), lambda i: (0, 0)),
            pl.BlockSpec((1, D), lambda i: (0, 0)),
        ],
        out_specs=pl.BlockSpec((blk, D), lambda i: (i, 0)),
        out_shape=jax.ShapeDtypeStruct((n_pad, D), jnp.float32),
    )(agg2, deg_t, feat_pad, wt, b2)


@jax.jit
def kernel(feature, edge_index, W, b):
    n = feature.shape[0]
    e = edge_index.shape[1]
    n_pad = ((n + 1 + NS * CHUNK - 1) // (NS * CHUNK)) * (NS * CHUNK)
    dummy = n

    src = edge_index[0].astype(jnp.int32)
    dst = edge_index[1].astype(jnp.int32)
    kch = -(-e // (NW * CHUNK))
    e_pad = NW * CHUNK * kch
    pad = e_pad - e
    src_w = jnp.concatenate(
        [src, jnp.full((pad,), dummy, jnp.int32)]).reshape(NW, kch, CHUNK)
    dst_w = jnp.concatenate(
        [dst, jnp.full((pad,), dummy, jnp.int32)]).reshape(NW, kch, CHUNK)

    feat_pad = jnp.zeros((n_pad, D), jnp.float32).at[:n].set(feature)
    zin = jnp.zeros((CHUNK, D), jnp.float32)

    agg2, deg_p = _sc_aggregate(feat_pad, src_w, dst_w, zin, n_pad)
    out_pad = _tc_dense(agg2, deg_p.T, feat_pad, W.T, b.reshape(1, D), n_pad)
    return out_pad[:n]
